# baseline (speedup 1.0000x reference)
import jax
import jax.numpy as jnp
from jax import lax
from jax.experimental import pallas as pl
from jax.experimental.pallas import tpu as pltpu
from jax.experimental.pallas import tpu_sc as plsc

_LANES = 16


def _sc_routing_kernel(idx_hbm, wgt_hbm, tvec_hbm, w_hbm,
                       idx_v, wgt_v, tvec_v, dest_v, zero_v, acc_v, acc_shared):
    cid = lax.axis_index("c")
    sid = lax.axis_index("s")
    num_pairs = idx_v.shape[0]
    num_tokens = tvec_v.shape[0] // 2

    @pl.when((cid == 0) & (sid == 0))
    def _():
        pltpu.sync_copy(idx_hbm, idx_v)
        pltpu.sync_copy(wgt_hbm, wgt_v)
        pltpu.sync_copy(tvec_hbm, tvec_v)
        for i in range(zero_v.shape[0] // _LANES):
            zero_v[pl.ds(i * _LANES, _LANES)] = jnp.zeros((_LANES,), jnp.float32)
        for i in range(num_pairs // _LANES):
            chunk_e = idx_v[pl.ds(i * _LANES, _LANES)]
            chunk_t = tvec_v[pl.ds(i * _LANES, _LANES)]
            dest_v[pl.ds(i * _LANES, _LANES)] = chunk_e * num_tokens + chunk_t
        pltpu.sync_copy(zero_v, acc_shared)
        pltpu.sync_copy(wgt_v, acc_shared.at[dest_v], add=True)
        pltpu.sync_copy(acc_shared, acc_v)
        pltpu.sync_copy(acc_v, w_hbm)


def _combine_map(expert_indices, expert_weights, num_experts):
    num_tokens, top_k = expert_indices.shape
    idx_flat = expert_indices.astype(jnp.int32).T.reshape(-1)
    wgt_flat = expert_weights.T.reshape(-1)
    tvec = jnp.tile(jnp.arange(num_tokens, dtype=jnp.int32), top_k)
    n = num_experts * num_tokens
    mesh = plsc.VectorSubcoreMesh(core_axis_name="c", subcore_axis_name="s")
    w_flat = pl.kernel(
        _sc_routing_kernel,
        out_type=jax.ShapeDtypeStruct((n,), jnp.float32),
        mesh=mesh,
        scratch_types=[
            pltpu.VMEM((idx_flat.shape[0],), jnp.int32),
            pltpu.VMEM((wgt_flat.shape[0],), jnp.float32),
            pltpu.VMEM((tvec.shape[0],), jnp.int32),
            pltpu.VMEM((idx_flat.shape[0],), jnp.int32),
            pltpu.VMEM((n,), jnp.float32),
            pltpu.VMEM((n,), jnp.float32),
            pltpu.MemorySpace.VMEM_SHARED((n,), jnp.float32),
        ],
    )(idx_flat, wgt_flat, tvec)
    return w_flat.reshape(num_experts, 1, num_tokens)


def _outer(x_ref, wmap_hbm, gup_hbm, down_hbm, out_ref):
    num_experts = gup_hbm.shape[0]
    interm = down_hbm.shape[1]
    out_ref[...] = jnp.zeros_like(out_ref)
    x = x_ref[...].astype(jnp.bfloat16)

    def body(w_blk, gup_blk, down_blk):
        gu = jnp.dot(x, gup_blk[0].astype(jnp.bfloat16),
                     preferred_element_type=jnp.float32)
        gate = gu[:, :interm]
        up = gu[:, interm:]
        act = (gate * jax.nn.sigmoid(gate) * up) * w_blk[0, 0, :][:, None]
        oe = jnp.dot(act.astype(jnp.bfloat16), down_blk[0].astype(jnp.bfloat16),
                     preferred_element_type=jnp.float32)
        out_ref[...] += oe

    pltpu.emit_pipeline(
        body,
        grid=(num_experts,),
        in_specs=[
            pl.BlockSpec((1, 1, wmap_hbm.shape[2]), lambda e: (e, 0, 0),
                         pipeline_mode=pl.Buffered(buffer_count=4)),
            pl.BlockSpec((1, gup_hbm.shape[1], gup_hbm.shape[2]),
                         lambda e: (e, 0, 0),
                         pipeline_mode=pl.Buffered(buffer_count=4)),
            pl.BlockSpec((1, interm, down_hbm.shape[2]), lambda e: (e, 0, 0),
                         pipeline_mode=pl.Buffered(buffer_count=4)),
        ],
    )(wmap_hbm, gup_hbm, down_hbm)


def kernel(hidden_states, gate_up_proj, down_proj, expert_indices, expert_weights):
    num_tokens, hidden = hidden_states.shape
    num_experts = gate_up_proj.shape[0]
    wmap = _combine_map(expert_indices, expert_weights, num_experts)

    return pl.pallas_call(
        _outer,
        in_specs=[
            pl.BlockSpec(memory_space=pltpu.MemorySpace.VMEM),
            pl.BlockSpec(memory_space=pltpu.MemorySpace.HBM),
            pl.BlockSpec(memory_space=pltpu.MemorySpace.HBM),
            pl.BlockSpec(memory_space=pltpu.MemorySpace.HBM),
        ],
        out_specs=pl.BlockSpec(memory_space=pltpu.MemorySpace.VMEM),
        out_shape=jax.ShapeDtypeStruct((num_tokens, hidden), jnp.float32),
    )(hidden_states, wmap, gate_up_proj, down_proj)


# speedup vs baseline: 1.6055x; 1.6055x over previous
import jax
import jax.numpy as jnp
from jax.experimental import pallas as pl
from jax.experimental.pallas import tpu as pltpu


def _outer(idx_ref, wgt_ref, x_ref, gup_hbm, down_hbm, out_ref):
    num_experts = gup_hbm.shape[0]
    hidden = gup_hbm.shape[1]
    interm = down_hbm.shape[1]
    out_ref[...] = jnp.zeros_like(out_ref)
    x = x_ref[...].astype(jnp.bfloat16)
    idx = idx_ref[...]
    wgt = wgt_ref[...]

    def body(gate_blk, up_blk, down_blk):
        e = pl.program_id(0)
        gate = jnp.dot(x, gate_blk[0].astype(jnp.bfloat16),
                       preferred_element_type=jnp.float32)
        up = jnp.dot(x, up_blk[0].astype(jnp.bfloat16),
                     preferred_element_type=jnp.float32)
        w = jnp.sum(jnp.where(idx == e, wgt, 0.0), axis=0)
        act = (gate * jax.nn.sigmoid(gate) * up) * w[:, None]
        oe = jnp.dot(act.astype(jnp.bfloat16), down_blk[0].astype(jnp.bfloat16),
                     preferred_element_type=jnp.float32)
        out_ref[...] += oe

    pltpu.emit_pipeline(
        body,
        grid=(num_experts,),
        in_specs=[
            pl.BlockSpec((1, hidden, interm), lambda e: (e, 0, 0),
                         pipeline_mode=pl.Buffered(buffer_count=6)),
            pl.BlockSpec((1, hidden, interm), lambda e: (e, 0, 1),
                         pipeline_mode=pl.Buffered(buffer_count=6)),
            pl.BlockSpec((1, interm, hidden), lambda e: (e, 0, 0),
                         pipeline_mode=pl.Buffered(buffer_count=6)),
        ],
    )(gup_hbm, gup_hbm, down_hbm)


def kernel(hidden_states, gate_up_proj, down_proj, expert_indices, expert_weights):
    num_tokens, hidden = hidden_states.shape
    idx_t = expert_indices.astype(jnp.int32).T
    wgt_t = expert_weights.T

    return pl.pallas_call(
        _outer,
        in_specs=[
            pl.BlockSpec(memory_space=pltpu.MemorySpace.VMEM),
            pl.BlockSpec(memory_space=pltpu.MemorySpace.VMEM),
            pl.BlockSpec(memory_space=pltpu.MemorySpace.VMEM),
            pl.BlockSpec(memory_space=pltpu.MemorySpace.HBM),
            pl.BlockSpec(memory_space=pltpu.MemorySpace.HBM),
        ],
        out_specs=pl.BlockSpec(memory_space=pltpu.MemorySpace.VMEM),
        out_shape=jax.ShapeDtypeStruct((num_tokens, hidden), jnp.float32),
    )(idx_t, wgt_t, hidden_states, gate_up_proj, down_proj)


# speedup vs baseline: 1.6190x; 1.0084x over previous
import jax
import jax.numpy as jnp
from jax.experimental import pallas as pl
from jax.experimental.pallas import tpu as pltpu


def _outer(idx_ref, wgt_ref, x_ref, gup_hbm, down_hbm, out_ref):
    num_experts = gup_hbm.shape[0]
    hidden = gup_hbm.shape[1]
    interm = down_hbm.shape[1]
    out_ref[...] = jnp.zeros_like(out_ref)
    x = x_ref[...].astype(jnp.bfloat16)
    idx = idx_ref[...]
    wgt = wgt_ref[...]

    def body(gate_blk, up_blk, down_blk):
        e = pl.program_id(0)
        gate = jnp.dot(x, gate_blk[0].astype(jnp.bfloat16),
                       preferred_element_type=jnp.float32)
        up = jnp.dot(x, up_blk[0].astype(jnp.bfloat16),
                     preferred_element_type=jnp.float32)
        w = jnp.sum(jnp.where(idx == e, wgt, 0.0), axis=0)
        act = (gate * jax.nn.sigmoid(gate) * up) * w[:, None]
        oe = jnp.dot(act.astype(jnp.bfloat16), down_blk[0].astype(jnp.bfloat16),
                     preferred_element_type=jnp.float32)
        out_ref[...] += oe

    pltpu.emit_pipeline(
        body,
        grid=(num_experts,),
        in_specs=[
            pl.BlockSpec((1, hidden, interm), lambda e: (e, 0, 0),
                         pipeline_mode=pl.Buffered(buffer_count=4)),
            pl.BlockSpec((1, hidden, interm), lambda e: (e, 0, 1),
                         pipeline_mode=pl.Buffered(buffer_count=4)),
            pl.BlockSpec((1, interm, hidden), lambda e: (e, 0, 0),
                         pipeline_mode=pl.Buffered(buffer_count=4)),
        ],
    )(gup_hbm, gup_hbm, down_hbm)


def kernel(hidden_states, gate_up_proj, down_proj, expert_indices, expert_weights):
    num_tokens, hidden = hidden_states.shape
    idx_t = expert_indices.astype(jnp.int32).T
    wgt_t = expert_weights.T

    return pl.pallas_call(
        _outer,
        in_specs=[
            pl.BlockSpec(memory_space=pltpu.MemorySpace.VMEM),
            pl.BlockSpec(memory_space=pltpu.MemorySpace.VMEM),
            pl.BlockSpec(memory_space=pltpu.MemorySpace.VMEM),
            pl.BlockSpec(memory_space=pltpu.MemorySpace.HBM),
            pl.BlockSpec(memory_space=pltpu.MemorySpace.HBM),
        ],
        out_specs=pl.BlockSpec(memory_space=pltpu.MemorySpace.VMEM),
        out_shape=jax.ShapeDtypeStruct((num_tokens, hidden), jnp.float32),
    )(idx_t, wgt_t, hidden_states, gate_up_proj, down_proj)
